# Initial kernel scaffold; baseline (speedup 1.0000x reference)
import functools

import jax
import jax.numpy as jnp
from jax import lax
from jax.experimental import pallas as pl
from jax.experimental.pallas import tpu as pltpu
from jax.experimental.pallas import tpu_sc as plsc

NC = 2
NS = 16
L = 16
NW = NC * NS

B = 16
S = 2048
H = 128
N = B * S
RPW = N // NW
C = 128
NCHUNK = RPW // C
HV = H // L


def _rsqrt16(x):
    i = plsc.bitcast(x, jnp.int32)
    i = jnp.int32(0x5F3759DF) - lax.shift_right_logical(i, jnp.int32(1))
    y = plsc.bitcast(i, jnp.float32)
    for _ in range(3):
        y = y * (jnp.float32(1.5) - jnp.float32(0.5) * x * y * y)
    return y


def _body(tok_hbm, idx_hbm, pos_hbm, gamma_hbm, beta_hbm, out_hbm,
          idx_v, tok_v, pos_v, out_v, gamma_v, beta_v,
          tok_sems, pos_sems, out_sems):
    wid = lax.axis_index("s") * NC + lax.axis_index("c")
    pos_base = (wid % 2) * RPW
    row_base = wid * RPW

    pltpu.sync_copy(idx_hbm.at[wid], idx_v)
    pltpu.sync_copy(gamma_hbm, gamma_v)
    pltpu.sync_copy(beta_hbm, beta_v)

    gvs = [gamma_v[pl.ds(L * j, L)] for j in range(HV)]
    bvs = [beta_v[pl.ds(L * j, L)] for j in range(HV)]

    def start(c):
        p = c % 2
        t = pltpu.async_copy(tok_hbm.at[idx_v.at[c]], tok_v.at[p],
                             tok_sems[p])
        q = pltpu.async_copy(pos_hbm.at[pl.ds(pos_base + c * C, C)],
                             pos_v.at[p], pos_sems[p])
        return (t, q)

    inflight = [None] * NCHUNK
    outflight = [None] * NCHUNK
    inflight[0] = start(0)

    for c in range(NCHUNK):
        p = c % 2
        if c + 1 < NCHUNK:
            inflight[c + 1] = start(c + 1)
        t, q = inflight[c]
        t.wait()
        q.wait()
        if c >= 2:
            outflight[c - 2].wait()

        def row(r, carry, p=p):
            xs = []
            for j in range(HV):
                xs.append(tok_v[p, r, pl.ds(L * j, L)]
                          + pos_v[p, r, pl.ds(L * j, L)])
            vsum = xs[0]
            vsq = xs[0] * xs[0]
            for j in range(1, HV):
                vsum = vsum + xs[j]
                vsq = vsq + xs[j] * xs[j]
            s = jnp.sum(vsum)
            ss = jnp.sum(vsq)
            mean = s * jnp.float32(1.0 / H)
            var = ss * jnp.float32(1.0 / H) - mean * mean
            mean_v = jnp.full((L,), mean, jnp.float32)
            rstd_v = _rsqrt16(jnp.full((L,), var + jnp.float32(1e-12),
                                       jnp.float32))
            for j in range(HV):
                out_v[p, r, pl.ds(L * j, L)] = (
                    (xs[j] - mean_v) * rstd_v * gvs[j] + bvs[j])
            return carry

        lax.fori_loop(0, C, row, 0)
        outflight[c] = pltpu.async_copy(
            out_v.at[p], out_hbm.at[pl.ds(row_base + c * C, C)],
            out_sems[p])

    outflight[NCHUNK - 2].wait()
    outflight[NCHUNK - 1].wait()


@jax.jit
def kernel(input_ids, token_table, position_table, gamma, beta):
    idx = input_ids.reshape(NW, NCHUNK, C)
    mesh = plsc.VectorSubcoreMesh(core_axis_name="c", subcore_axis_name="s",
                                  num_cores=NC, num_subcores=NS)
    out = pl.kernel(
        _body,
        out_type=jax.ShapeDtypeStruct((N, H), jnp.float32),
        mesh=mesh,
        scratch_types=[
            pltpu.VMEM((NCHUNK, C), jnp.int32),
            pltpu.VMEM((2, C, H), jnp.float32),
            pltpu.VMEM((2, C, H), jnp.float32),
            pltpu.VMEM((2, C, H), jnp.float32),
            pltpu.VMEM((H,), jnp.float32),
            pltpu.VMEM((H,), jnp.float32),
            [pltpu.SemaphoreType.DMA, pltpu.SemaphoreType.DMA],
            [pltpu.SemaphoreType.DMA, pltpu.SemaphoreType.DMA],
            [pltpu.SemaphoreType.DMA, pltpu.SemaphoreType.DMA],
        ],
    )(token_table, idx, position_table, gamma, beta)
    return out.reshape(B, S, H)


# speedup vs baseline: 1.5198x; 1.5198x over previous
import functools

import jax
import jax.numpy as jnp
from jax import lax
from jax.experimental import pallas as pl
from jax.experimental.pallas import tpu as pltpu
from jax.experimental.pallas import tpu_sc as plsc

NC = 2
NS = 16
L = 16
NW = NC * NS

B = 16
S = 2048
H = 128
N = B * S
RPW = N // NW
C = 128
NCHUNK = RPW // C
HV = H // L


def _perm16(v, idx):
    dn = lax.GatherDimensionNumbers(offset_dims=(), collapsed_slice_dims=(0,),
                                    start_index_map=(0,))
    return lax.gather(v, idx[:, None], dn, (1,),
                      mode=lax.GatherScatterMode.PROMISE_IN_BOUNDS)


def _allreduce_sum16(v):
    base = lax.iota(jnp.int32, L)
    for sh in (8, 4, 2, 1):
        v = v + _perm16(v, base ^ sh)
    return v


def _rsqrt16(x):
    i = plsc.bitcast(x, jnp.int32)
    i = jnp.int32(0x5F3759DF) - lax.shift_right_logical(i, jnp.int32(1))
    y = plsc.bitcast(i, jnp.float32)
    for _ in range(3):
        y = y * (jnp.float32(1.5) - jnp.float32(0.5) * x * y * y)
    return y


def _body(tok_hbm, idx_hbm, pos_hbm, gamma_hbm, beta_hbm, out_hbm,
          idx_v, tok_v, pos_v, out_v, gamma_v, beta_v,
          tok_sems, pos_sems, out_sems):
    wid = lax.axis_index("s") * NC + lax.axis_index("c")
    pos_base = (wid % 2) * RPW
    row_base = wid * RPW

    pltpu.sync_copy(idx_hbm.at[wid], idx_v)
    pltpu.sync_copy(gamma_hbm, gamma_v)
    pltpu.sync_copy(beta_hbm, beta_v)

    gvs = [gamma_v[pl.ds(L * j, L)] for j in range(HV)]
    bvs = [beta_v[pl.ds(L * j, L)] for j in range(HV)]

    def start(c):
        p = c % 2
        t = pltpu.async_copy(tok_hbm.at[idx_v.at[c]], tok_v.at[p],
                             tok_sems[p])
        q = pltpu.async_copy(pos_hbm.at[pl.ds(pos_base + c * C, C)],
                             pos_v.at[p], pos_sems[p])
        return (t, q)

    inflight = [None] * NCHUNK
    outflight = [None] * NCHUNK
    inflight[0] = start(0)

    for c in range(NCHUNK):
        p = c % 2
        if c + 1 < NCHUNK:
            inflight[c + 1] = start(c + 1)
        t, q = inflight[c]
        t.wait()
        q.wait()
        if c >= 2:
            outflight[c - 2].wait()

        def row(r, carry, p=p):
            xs = []
            for j in range(HV):
                xs.append(tok_v[p, r, pl.ds(L * j, L)]
                          + pos_v[p, r, pl.ds(L * j, L)])
            vsum = xs[0]
            vsq = xs[0] * xs[0]
            for j in range(1, HV):
                vsum = vsum + xs[j]
                vsq = vsq + xs[j] * xs[j]
            mean_v = _allreduce_sum16(vsum) * jnp.float32(1.0 / H)
            var_v = (_allreduce_sum16(vsq) * jnp.float32(1.0 / H)
                     - mean_v * mean_v)
            rstd_v = _rsqrt16(var_v + jnp.float32(1e-12))
            for j in range(HV):
                out_v[p, r, pl.ds(L * j, L)] = (
                    (xs[j] - mean_v) * rstd_v * gvs[j] + bvs[j])
            return carry

        lax.fori_loop(0, C, row, 0)
        outflight[c] = pltpu.async_copy(
            out_v.at[p], out_hbm.at[pl.ds(row_base + c * C, C)],
            out_sems[p])

    outflight[NCHUNK - 2].wait()
    outflight[NCHUNK - 1].wait()


@jax.jit
def kernel(input_ids, token_table, position_table, gamma, beta):
    idx = input_ids.reshape(NW, NCHUNK, C)
    mesh = plsc.VectorSubcoreMesh(core_axis_name="c", subcore_axis_name="s",
                                  num_cores=NC, num_subcores=NS)
    out = pl.kernel(
        _body,
        out_type=jax.ShapeDtypeStruct((N, H), jnp.float32),
        mesh=mesh,
        compiler_params=pltpu.CompilerParams(needs_layout_passes=False),
        scratch_types=[
            pltpu.VMEM((NCHUNK, C), jnp.int32),
            pltpu.VMEM((2, C, H), jnp.float32),
            pltpu.VMEM((2, C, H), jnp.float32),
            pltpu.VMEM((2, C, H), jnp.float32),
            pltpu.VMEM((H,), jnp.float32),
            pltpu.VMEM((H,), jnp.float32),
            [pltpu.SemaphoreType.DMA, pltpu.SemaphoreType.DMA],
            [pltpu.SemaphoreType.DMA, pltpu.SemaphoreType.DMA],
            [pltpu.SemaphoreType.DMA, pltpu.SemaphoreType.DMA],
        ],
    )(token_table, idx, position_table, gamma, beta)
    return out.reshape(B, S, H)


# speedup vs baseline: 1.5741x; 1.0358x over previous
import jax
import jax.numpy as jnp
from jax import lax
from jax.experimental import pallas as pl
from jax.experimental.pallas import tpu as pltpu
from jax.experimental.pallas import tpu_sc as plsc

NC = 2
NS = 16
L = 16
NW = NC * NS

B = 16
S = 2048
H = 128
C = S // NW
HV = H // L


def _perm16(v, idx):
    dn = lax.GatherDimensionNumbers(offset_dims=(), collapsed_slice_dims=(0,),
                                    start_index_map=(0,))
    return lax.gather(v, idx[:, None], dn, (1,),
                      mode=lax.GatherScatterMode.PROMISE_IN_BOUNDS)


def _allreduce_sum16(v):
    base = lax.iota(jnp.int32, L)
    for sh in (8, 4, 2, 1):
        v = v + _perm16(v, base ^ sh)
    return v


def _rsqrt16(x):
    i = plsc.bitcast(x, jnp.int32)
    i = jnp.int32(0x5F3759DF) - lax.shift_right_logical(i, jnp.int32(1))
    y = plsc.bitcast(i, jnp.float32)
    for _ in range(3):
        y = y * (jnp.float32(1.5) - jnp.float32(0.5) * x * y * y)
    return y


def _body(tok_hbm, idx_hbm, pos_hbm, gamma_hbm, beta_hbm, out_hbm,
          idx_v, tok_v, pos_v, out_v, gamma_v, beta_v,
          tok_sems, out_sems):
    wid = lax.axis_index("s") * NC + lax.axis_index("c")
    seq_base = wid * C

    pltpu.sync_copy(idx_hbm.at[wid], idx_v)
    pltpu.sync_copy(pos_hbm.at[pl.ds(seq_base, C)], pos_v)
    pltpu.sync_copy(gamma_hbm, gamma_v)
    pltpu.sync_copy(beta_hbm, beta_v)

    gvs = [gamma_v[pl.ds(L * j, L)] for j in range(HV)]
    bvs = [beta_v[pl.ds(L * j, L)] for j in range(HV)]

    def start(b):
        return pltpu.async_copy(tok_hbm.at[idx_v.at[b]], tok_v.at[b % 2],
                                tok_sems[b % 2])

    inflight = [None] * B
    outflight = [None] * B
    inflight[0] = start(0)

    for b in range(B):
        p = b % 2
        if b + 1 < B:
            inflight[b + 1] = start(b + 1)
        inflight[b].wait()
        if b >= 2:
            outflight[b - 2].wait()

        def row(r, carry, p=p):
            xs = []
            for j in range(HV):
                xs.append(tok_v[p, r, pl.ds(L * j, L)]
                          + pos_v[r, pl.ds(L * j, L)])
            vsum = xs[0]
            vsq = xs[0] * xs[0]
            for j in range(1, HV):
                vsum = vsum + xs[j]
                vsq = vsq + xs[j] * xs[j]
            mean_v = _allreduce_sum16(vsum) * jnp.float32(1.0 / H)
            var_v = (_allreduce_sum16(vsq) * jnp.float32(1.0 / H)
                     - mean_v * mean_v)
            rstd_v = _rsqrt16(var_v + jnp.float32(1e-12))
            for j in range(HV):
                out_v[p, r, pl.ds(L * j, L)] = (
                    (xs[j] - mean_v) * rstd_v * gvs[j] + bvs[j])
            return carry

        lax.fori_loop(0, C, row, 0)
        outflight[b] = pltpu.async_copy(
            out_v.at[p], out_hbm.at[pl.ds(b * S + seq_base, C)],
            out_sems[p])

    outflight[B - 2].wait()
    outflight[B - 1].wait()


@jax.jit
def kernel(input_ids, token_table, position_table, gamma, beta):
    idx = input_ids.reshape(B, NW, C).transpose(1, 0, 2)
    mesh = plsc.VectorSubcoreMesh(core_axis_name="c", subcore_axis_name="s",
                                  num_cores=NC, num_subcores=NS)
    out = pl.kernel(
        _body,
        out_type=jax.ShapeDtypeStruct((B * S, H), jnp.float32),
        mesh=mesh,
        compiler_params=pltpu.CompilerParams(needs_layout_passes=False),
        scratch_types=[
            pltpu.VMEM((B, C), jnp.int32),
            pltpu.VMEM((2, C, H), jnp.float32),
            pltpu.VMEM((C, H), jnp.float32),
            pltpu.VMEM((2, C, H), jnp.float32),
            pltpu.VMEM((H,), jnp.float32),
            pltpu.VMEM((H,), jnp.float32),
            [pltpu.SemaphoreType.DMA, pltpu.SemaphoreType.DMA],
            [pltpu.SemaphoreType.DMA, pltpu.SemaphoreType.DMA],
        ],
    )(token_table, idx, position_table, gamma, beta)
    return out.reshape(B, S, H)
